# baseline (speedup 1.0000x reference)
import jax
import jax.numpy as jnp
from jax.experimental import pallas as pl
from jax.experimental.pallas import tpu as pltpu


def _pp_kernel(idx_ref, logits_ref, boxes_ref, scale_ref, bias_ref, tri_ref, out_ref):
    x = logits_ref[0]
    n_q = x.shape[0]

    rowmax = jnp.max(x, axis=1, keepdims=True)
    denom = jnp.sum(jnp.exp(x - rowmax), axis=1, keepdims=True)
    masked = x + bias_ref[0]
    mmax = jnp.max(masked, axis=1, keepdims=True)
    s_col = jnp.exp(mmax - rowmax) / denom

    a_col = jax.lax.bitcast_convert_type(s_col, jnp.int32) * 2
    a_row = jnp.transpose(a_col)
    m_cnt = jnp.where((a_col + tri_ref[...]) > a_row, 1, 0)
    rank_row = jnp.sum(m_cnt, axis=0, keepdims=True)
    iota_m = jax.lax.broadcasted_iota(jnp.int32, (n_q, n_q), 0)

    onehot = (iota_m == rank_row).astype(jnp.bfloat16)

    bx = boxes_ref[0]
    cx, cy, w, h = bx[:, 0:1], bx[:, 1:2], bx[:, 2:3], bx[:, 3:4]
    xyxy = jnp.concatenate(
        [cx - 0.5 * w, cy - 0.5 * h, cx + 0.5 * w, cy + 0.5 * h], axis=1
    )
    xyxy = xyxy * scale_ref[0]

    hi = xyxy.astype(jnp.bfloat16)
    r1 = xyxy - hi.astype(jnp.float32)
    mid = r1.astype(jnp.bfloat16)
    lo = (r1 - mid.astype(jnp.float32)).astype(jnp.bfloat16)
    rhs = jnp.concatenate([hi, mid, lo], axis=1)
    y = jax.lax.dot(onehot, rhs, preferred_element_type=jnp.float32)
    out_ref[0] = (y[:, 0:4] + y[:, 4:8]) + y[:, 8:12]


def kernel(pred_logits, pred_boxes, target_sizes, positive_map, phrase_batch_idx):
    B, N, L = pred_logits.shape
    P = positive_map.shape[0]

    img_h = target_sizes[:, 0]
    img_w = target_sizes[:, 1]
    scale = jnp.stack([img_w, img_h, img_w, img_h], axis=1).reshape(B, 1, 4)
    bias = jnp.where(positive_map > 1e-6, 0.0, -jnp.inf).astype(jnp.float32)
    bias = bias.reshape(P, 1, L)
    row_i = jax.lax.broadcasted_iota(jnp.int32, (N, N), 0)
    col_i = jax.lax.broadcasted_iota(jnp.int32, (N, N), 1)
    tri = (row_i < col_i).astype(jnp.int32)

    grid_spec = pltpu.PrefetchScalarGridSpec(
        num_scalar_prefetch=1,
        grid=(P,),
        in_specs=[
            pl.BlockSpec((1, N, L), lambda p, idx: (idx[p], 0, 0)),
            pl.BlockSpec((1, N, 4), lambda p, idx: (idx[p], 0, 0)),
            pl.BlockSpec((1, 1, 4), lambda p, idx: (idx[p], 0, 0)),
            pl.BlockSpec((1, 1, L), lambda p, idx: (p, 0, 0)),
            pl.BlockSpec((N, N), lambda p, idx: (0, 0)),
        ],
        out_specs=pl.BlockSpec((1, N, 4), lambda p, idx: (p, 0, 0)),
    )
    return pl.pallas_call(
        _pp_kernel,
        grid_spec=grid_spec,
        out_shape=jax.ShapeDtypeStruct((P, N, 4), jnp.float32),
        compiler_params=pltpu.CompilerParams(
            dimension_semantics=("parallel",),
        ),
    )(phrase_batch_idx, pred_logits, pred_boxes, scale, bias, tri)


# speedup vs baseline: 1.8893x; 1.8893x over previous
import jax
import jax.numpy as jnp
from jax.experimental import pallas as pl
from jax.experimental.pallas import tpu as pltpu


def _pp_kernel(offs_ref, logits_ref, boxes_ref, scale_ref, bias_ref, tri_ref,
               out_ref):
    b = pl.program_id(0)
    x = logits_ref[0]
    n_q = x.shape[0]

    rowmax = jnp.max(x, axis=1, keepdims=True)
    denom = jnp.sum(jnp.exp(x - rowmax), axis=1, keepdims=True)

    bx = boxes_ref[0]
    cx, cy, w, h = bx[0:1], bx[1:2], bx[2:3], bx[3:4]
    xyxy = jnp.concatenate(
        [cx - 0.5 * w, cy - 0.5 * h, cx + 0.5 * w, cy + 0.5 * h], axis=0
    ) * scale_ref[0]
    hi = xyxy.astype(jnp.bfloat16)
    r1 = xyxy - hi.astype(jnp.float32)
    mid = r1.astype(jnp.bfloat16)
    lo = (r1 - mid.astype(jnp.float32)).astype(jnp.bfloat16)
    rhs = jnp.concatenate([hi, mid, lo], axis=0)

    iota_j = jax.lax.broadcasted_iota(jnp.int32, (n_q, n_q), 1)
    tri = tri_ref[...]

    def body(p, carry):
        masked = x + bias_ref[p]
        mmax = jnp.max(masked, axis=1, keepdims=True)
        s_col = jnp.exp(mmax - rowmax) / denom
        a_col = jax.lax.bitcast_convert_type(s_col, jnp.int32) * 2
        a_row = jnp.transpose(a_col)
        m_cnt = jnp.where((a_row + tri) > a_col, 1, 0)
        rank_col = jnp.sum(m_cnt, axis=1, keepdims=True)
        onehot = (iota_j == rank_col).astype(jnp.bfloat16)
        y = jax.lax.dot(rhs, onehot, preferred_element_type=jnp.float32)
        out_ref[p] = (y[0:4] + y[4:8]) + y[8:12]
        return carry

    jax.lax.fori_loop(offs_ref[b], offs_ref[b + 1], body, 0)


def kernel(pred_logits, pred_boxes, target_sizes, positive_map, phrase_batch_idx):
    B, N, L = pred_logits.shape
    P = positive_map.shape[0]

    boxes_t = jnp.transpose(pred_boxes, (0, 2, 1))
    img_h = target_sizes[:, 0]
    img_w = target_sizes[:, 1]
    scale = jnp.stack([img_w, img_h, img_w, img_h], axis=1).reshape(B, 4, 1)
    bias = jnp.where(positive_map > 1e-6, 0.0, -jnp.inf).astype(jnp.float32)
    bias = bias.reshape(P, 1, L)
    row_i = jax.lax.broadcasted_iota(jnp.int32, (N, N), 0)
    col_i = jax.lax.broadcasted_iota(jnp.int32, (N, N), 1)
    tri = (col_i < row_i).astype(jnp.int32)
    offs = jnp.searchsorted(
        phrase_batch_idx, jnp.arange(B + 1, dtype=jnp.int32)
    ).astype(jnp.int32)

    grid_spec = pltpu.PrefetchScalarGridSpec(
        num_scalar_prefetch=1,
        grid=(B,),
        in_specs=[
            pl.BlockSpec((1, N, L), lambda b, offs: (b, 0, 0)),
            pl.BlockSpec((1, 4, N), lambda b, offs: (b, 0, 0)),
            pl.BlockSpec((1, 4, 1), lambda b, offs: (b, 0, 0)),
            pl.BlockSpec((P, 1, L), lambda b, offs: (0, 0, 0)),
            pl.BlockSpec((N, N), lambda b, offs: (0, 0)),
        ],
        out_specs=pl.BlockSpec((P, 4, N), lambda b, offs: (0, 0, 0)),
    )
    out_t = pl.pallas_call(
        _pp_kernel,
        grid_spec=grid_spec,
        out_shape=jax.ShapeDtypeStruct((P, 4, N), jnp.float32),
    )(offs, pred_logits, boxes_t, scale, bias, tri)
    return jnp.transpose(out_t, (0, 2, 1))
